# baseline (speedup 1.0000x reference)
import functools

import jax
import jax.numpy as jnp
from jax import lax
from jax.experimental import pallas as pl
from jax.experimental.pallas import tpu as pltpu
from jax.experimental.pallas import tpu_sc as plsc

N = 10000
D_IN = 128
D_OUT = 16
E = 320000

NC = 2
NS = 16
NW = NC * NS
LANES = 16
NPAD = 10240
RPS = NPAD // NS
EPS = E // NS
EPW = E // NW
CH = 1000
KCH = EPW // CH
NBUF = 2

_mesh = plsc.VectorSubcoreMesh(
    core_axis_name="c", subcore_axis_name="s", num_cores=NC, num_subcores=NS
)
_CP2 = pltpu.CompilerParams(needs_layout_passes=False, use_tc_tiling_on_sc=False)


def _mm_body(x_ref, w_ref, h_ref):
    h_ref[...] = jnp.dot(x_ref[...], w_ref[...],
                         preferred_element_type=jnp.float32)


def _mm_call(x, w):
    blk = 1024
    grid = NPAD // blk
    return pl.pallas_call(
        _mm_body,
        grid=(grid,),
        in_specs=[
            pl.BlockSpec((blk, D_IN), lambda i: (i, 0)),
            pl.BlockSpec((D_IN, D_OUT), lambda i: (0, 0)),
        ],
        out_specs=pl.BlockSpec((blk, D_OUT), lambda i: (i, 0)),
        out_shape=jax.ShapeDtypeStruct((NPAD, D_OUT), jnp.float32),
    )(x, w)


def _newton_rsqrt(x):
    i = plsc.bitcast(x, jnp.int32)
    i = jnp.int32(0x5F3759DF) - lax.shift_right_arithmetic(i, 1)
    y = plsc.bitcast(i, jnp.float32)
    xh = x * 0.5
    for _ in range(3):
        y = y * (1.5 - xh * y * y)
    return y


@functools.partial(
    pl.kernel,
    out_type=(
        jax.ShapeDtypeStruct((NC, NPAD, D_OUT), jnp.float32),
        jax.ShapeDtypeStruct((NC, NPAD), jnp.float32),
        jax.ShapeDtypeStruct((NPAD, D_OUT), jnp.float32),
    ),
    mesh=_mesh,
    compiler_params=_CP2,
    scratch_types=[
        pltpu.VMEM((EPS,), jnp.int32),
        pltpu.VMEM((NPAD,), jnp.float32),
        pltpu.VMEM((NS, RPS), jnp.float32),
        pltpu.VMEM((RPS,), jnp.float32),
        pltpu.VMEM((RPS,), jnp.float32),
        pltpu.VMEM((RPS, D_OUT), jnp.float32),
        pltpu.VMEM((EPW,), jnp.int32),
        pltpu.VMEM((EPW,), jnp.int32),
        pltpu.VMEM((NBUF, CH, D_OUT), jnp.float32),
        pltpu.VMEM_SHARED((NS, NPAD), jnp.float32),
        pltpu.VMEM_SHARED((NPAD, D_OUT), jnp.float32),
        pltpu.SemaphoreType.DMA,
        pltpu.SemaphoreType.DMA,
    ],
)
def _sc_kernel(dst_hbm, src_hbm, h_hbm,
               acc_hbm, deg_hbm, g_hbm,
               cidx_v, hist_v, tmp_v, deg_v, dinv_v, buf_v,
               src_v, dstc_v, rows_v, pub_sh, acc_sh, sem0, sem1):
    cid = lax.axis_index("c")
    sid = lax.axis_index("s")
    wid = cid * NS + sid
    base = sid * RPS
    sems = (sem0, sem1)

    pltpu.sync_copy(dst_hbm.at[pl.ds(sid * EPS, EPS)], cidx_v)

    def _zero_hist(i, carry):
        hist_v[pl.ds(i * LANES, LANES)] = jnp.zeros((LANES,), jnp.float32)
        return carry
    lax.fori_loop(0, NPAD // LANES, _zero_hist, 0, unroll=8)

    ones = jnp.ones((LANES,), jnp.float32)

    def _count(i, carry):
        idx = cidx_v[pl.ds(i * LANES, LANES)]
        plsc.addupdate_scatter(hist_v, [idx], ones)
        return carry
    lax.fori_loop(0, EPS // LANES, _count, 0, unroll=8)

    pltpu.sync_copy(hist_v, pub_sh.at[sid])

    pltpu.sync_copy(src_hbm.at[pl.ds(wid * EPW, EPW)], src_v)
    pltpu.sync_copy(dst_hbm.at[pl.ds(wid * EPW, EPW)], dstc_v)

    def _zero_buf(i, carry):
        buf_v[i] = jnp.zeros((D_OUT,), jnp.float32)
        return carry
    lax.fori_loop(0, RPS, _zero_buf, 0, unroll=8)
    pltpu.sync_copy(buf_v, acc_sh.at[pl.ds(base, RPS)])

    plsc.subcore_barrier()

    pltpu.sync_copy(pub_sh.at[:, pl.ds(base, RPS)], tmp_v)

    def _comb(j, carry):
        sl = pl.ds(j * LANES, LANES)
        v = tmp_v[0, sl]
        for t in range(1, NS):
            v = v + tmp_v[t, sl]
        deg_v[sl] = v + 1.0
        dinv_v[sl] = _newton_rsqrt(v + 1.0)
        return carry
    lax.fori_loop(0, RPS // LANES, _comb, 0, unroll=4)

    pltpu.sync_copy(deg_v, deg_hbm.at[cid, pl.ds(base, RPS)])

    pltpu.sync_copy(h_hbm.at[pl.ds(base, RPS)], buf_v)

    def _scale(r, carry):
        w = plsc.load_gather(dinv_v, [jnp.full((LANES,), r, jnp.int32)])
        buf_v[r] = buf_v[r] * w
        return carry
    lax.fori_loop(0, RPS, _scale, 0, unroll=8)

    pltpu.sync_copy(buf_v, g_hbm.at[pl.ds(base, RPS)])
    plsc.subcore_barrier()

    pltpu.async_copy(g_hbm.at[src_v.at[pl.ds(0, CH)]], rows_v.at[0], sem0)

    def _chunk(k, carry):
        p = lax.rem(k, NBUF)

        @pl.when(k + 1 < KCH)
        def _():
            pn = lax.rem(k + 1, NBUF)
            for q in range(NBUF):
                @pl.when(pn == q)
                def _():
                    pltpu.async_copy(
                        g_hbm.at[src_v.at[pl.ds((k + 1) * CH, CH)]],
                        rows_v.at[q], sems[q])

        for q in range(NBUF):
            @pl.when(p == q)
            def _():
                pltpu.make_async_copy(
                    g_hbm.at[src_v.at[pl.ds(k * CH, CH)]],
                    rows_v.at[q], sems[q]).wait()
                pltpu.sync_copy(rows_v.at[q],
                                acc_sh.at[dstc_v.at[pl.ds(k * CH, CH)]],
                                add=True)
        return carry
    lax.fori_loop(0, KCH, _chunk, 0)

    plsc.subcore_barrier()
    pltpu.sync_copy(acc_sh.at[pl.ds(base, RPS)], buf_v)
    pltpu.sync_copy(buf_v, acc_hbm.at[cid, pl.ds(base, RPS), :])


def _fin_body(accp_ref, g_ref, degp_ref, b_ref, out_ref):
    deg = degp_ref[0, :, :]
    s = accp_ref[0, :, :] + accp_ref[1, :, :] + g_ref[...]
    out_ref[...] = s * lax.rsqrt(deg) + b_ref[0, :]


def _fin_call(accp, g, degp, b):
    blk = 1024
    grid = NPAD // blk
    return pl.pallas_call(
        _fin_body,
        grid=(grid,),
        in_specs=[
            pl.BlockSpec((NC, blk, D_OUT), lambda i: (0, i, 0)),
            pl.BlockSpec((blk, D_OUT), lambda i: (i, 0)),
            pl.BlockSpec((NC, blk, 1), lambda i: (0, i, 0)),
            pl.BlockSpec((1, D_OUT), lambda i: (0, 0)),
        ],
        out_specs=pl.BlockSpec((blk, D_OUT), lambda i: (i, 0)),
        out_shape=jax.ShapeDtypeStruct((N, D_OUT), jnp.float32),
    )(accp, g, degp, b)


def kernel(x, edge_index, W, b):
    src = edge_index[0]
    dst = edge_index[1]

    h = _mm_call(x, W)
    accp, degp, g = _sc_kernel(dst, src, h)
    out = _fin_call(accp, g, degp.reshape(NC, NPAD, 1), b.reshape(1, D_OUT))
    return out


# speedup vs baseline: 1.1674x; 1.1674x over previous
import functools

import jax
import jax.numpy as jnp
from jax import lax
from jax.experimental import pallas as pl
from jax.experimental.pallas import tpu as pltpu
from jax.experimental.pallas import tpu_sc as plsc

N = 10000
D_IN = 128
D_OUT = 16
E = 320000

NC = 2
NS = 16
NW = NC * NS
LANES = 16
NPAD = 10240
RPS = NPAD // NS
EPS = E // NS
EPW = E // NW
CH = 1000
KCH = EPW // CH
NBUF = 2

_mesh = plsc.VectorSubcoreMesh(
    core_axis_name="c", subcore_axis_name="s", num_cores=NC, num_subcores=NS
)
_CP2 = pltpu.CompilerParams(needs_layout_passes=False, use_tc_tiling_on_sc=False)


def _mm_body(x_ref, w_ref, h_ref):
    h_ref[...] = jnp.dot(x_ref[...], w_ref[...],
                         preferred_element_type=jnp.float32)


def _mm_call(x, w):
    blk = 1024
    grid = NPAD // blk
    return pl.pallas_call(
        _mm_body,
        grid=(grid,),
        in_specs=[
            pl.BlockSpec((blk, D_IN), lambda i: (i, 0)),
            pl.BlockSpec((D_IN, D_OUT), lambda i: (0, 0)),
        ],
        out_specs=pl.BlockSpec((blk, D_OUT), lambda i: (i, 0)),
        out_shape=jax.ShapeDtypeStruct((NPAD, D_OUT), jnp.float32),
    )(x, w)


def _newton_rsqrt(x):
    i = plsc.bitcast(x, jnp.int32)
    i = jnp.int32(0x5F3759DF) - lax.shift_right_arithmetic(i, 1)
    y = plsc.bitcast(i, jnp.float32)
    xh = x * 0.5
    for _ in range(3):
        y = y * (1.5 - xh * y * y)
    return y


@functools.partial(
    pl.kernel,
    out_type=(
        jax.ShapeDtypeStruct((NC, NPAD, D_OUT), jnp.float32),
        jax.ShapeDtypeStruct((NPAD, D_OUT), jnp.float32),
        jax.ShapeDtypeStruct((NPAD, D_OUT), jnp.float32),
    ),
    mesh=_mesh,
    compiler_params=_CP2,
    scratch_types=[
        pltpu.VMEM((EPS,), jnp.int32),
        pltpu.VMEM((NPAD,), jnp.float32),
        pltpu.VMEM((NS, RPS), jnp.float32),
        pltpu.VMEM((RPS,), jnp.float32),
        pltpu.VMEM((RPS, D_OUT), jnp.float32),
        pltpu.VMEM((EPW,), jnp.int32),
        pltpu.VMEM((EPW,), jnp.int32),
        pltpu.VMEM((NBUF, CH, D_OUT), jnp.float32),
        pltpu.VMEM_SHARED((NS, NPAD), jnp.float32),
        pltpu.VMEM_SHARED((NPAD, D_OUT), jnp.float32),
        pltpu.SemaphoreType.DMA,
        pltpu.SemaphoreType.DMA,
    ],
)
def _sc_kernel(edge_hbm, h_hbm,
               acc_hbm, dinvb_hbm, g_hbm,
               cidx_v, hist_v, tmp_v, dinv_v, buf_v,
               src_v, dstc_v, rows_v, pub_sh, acc_sh, sem0, sem1):
    cid = lax.axis_index("c")
    sid = lax.axis_index("s")
    wid = cid * NS + sid
    base = sid * RPS
    sems = (sem0, sem1)

    pltpu.sync_copy(edge_hbm.at[1, pl.ds(sid * EPS, EPS)], cidx_v)

    def _zero_hist(i, carry):
        hist_v[pl.ds(i * LANES, LANES)] = jnp.zeros((LANES,), jnp.float32)
        return carry
    lax.fori_loop(0, NPAD // LANES, _zero_hist, 0, unroll=8)

    ones = jnp.ones((LANES,), jnp.float32)

    def _count(i, carry):
        idx = cidx_v[pl.ds(i * LANES, LANES)]
        plsc.addupdate_scatter(hist_v, [idx], ones)
        return carry
    lax.fori_loop(0, EPS // LANES, _count, 0, unroll=8)

    pltpu.sync_copy(hist_v, pub_sh.at[sid])

    pltpu.sync_copy(edge_hbm.at[0, pl.ds(wid * EPW, EPW)], src_v)
    pltpu.sync_copy(edge_hbm.at[1, pl.ds(wid * EPW, EPW)], dstc_v)

    def _zero_buf(i, carry):
        buf_v[i] = jnp.zeros((D_OUT,), jnp.float32)
        return carry
    lax.fori_loop(0, RPS, _zero_buf, 0, unroll=8)
    pltpu.sync_copy(buf_v, acc_sh.at[pl.ds(base, RPS)])

    plsc.subcore_barrier()

    pltpu.sync_copy(pub_sh.at[:, pl.ds(base, RPS)], tmp_v)

    def _comb(j, carry):
        sl = pl.ds(j * LANES, LANES)
        v = tmp_v[0, sl]
        for t in range(1, NS):
            v = v + tmp_v[t, sl]
        dinv_v[sl] = _newton_rsqrt(v + 1.0)
        return carry
    lax.fori_loop(0, RPS // LANES, _comb, 0, unroll=4)

    @pl.when(cid == 0)
    def _():
        def _dbc(r, carry):
            w = plsc.load_gather(dinv_v, [jnp.full((LANES,), r, jnp.int32)])
            rows_v[0, r] = w
            return carry
        lax.fori_loop(0, RPS, _dbc, 0, unroll=8)
        pltpu.sync_copy(rows_v.at[0, pl.ds(0, RPS)],
                        dinvb_hbm.at[pl.ds(base, RPS)])

    pltpu.sync_copy(h_hbm.at[pl.ds(base, RPS)], buf_v)

    def _scale(r, carry):
        w = plsc.load_gather(dinv_v, [jnp.full((LANES,), r, jnp.int32)])
        buf_v[r] = buf_v[r] * w
        return carry
    lax.fori_loop(0, RPS, _scale, 0, unroll=8)

    pltpu.sync_copy(buf_v, g_hbm.at[pl.ds(base, RPS)])
    plsc.subcore_barrier()

    pltpu.async_copy(g_hbm.at[src_v.at[pl.ds(0, CH)]], rows_v.at[0], sem0)

    def _chunk(k, carry):
        p = lax.rem(k, NBUF)

        @pl.when(k + 1 < KCH)
        def _():
            pn = lax.rem(k + 1, NBUF)
            for q in range(NBUF):
                @pl.when(pn == q)
                def _():
                    pltpu.async_copy(
                        g_hbm.at[src_v.at[pl.ds((k + 1) * CH, CH)]],
                        rows_v.at[q], sems[q])

        for q in range(NBUF):
            @pl.when(p == q)
            def _():
                pltpu.make_async_copy(
                    g_hbm.at[src_v.at[pl.ds(k * CH, CH)]],
                    rows_v.at[q], sems[q]).wait()
                pltpu.sync_copy(rows_v.at[q],
                                acc_sh.at[dstc_v.at[pl.ds(k * CH, CH)]],
                                add=True)
        return carry
    lax.fori_loop(0, KCH, _chunk, 0)

    plsc.subcore_barrier()
    pltpu.sync_copy(acc_sh.at[pl.ds(base, RPS)], buf_v)
    pltpu.sync_copy(buf_v, acc_hbm.at[cid, pl.ds(base, RPS), :])


def _fin_body(accp_ref, g_ref, dinvb_ref, b_ref, out_ref):
    s = accp_ref[0, :, :] + accp_ref[1, :, :] + g_ref[...]
    out_ref[...] = s * dinvb_ref[...] + b_ref[0, :]


def _fin_call(accp, g, dinvb, b):
    blk = 1024
    grid = NPAD // blk
    return pl.pallas_call(
        _fin_body,
        grid=(grid,),
        in_specs=[
            pl.BlockSpec((NC, blk, D_OUT), lambda i: (0, i, 0)),
            pl.BlockSpec((blk, D_OUT), lambda i: (i, 0)),
            pl.BlockSpec((blk, D_OUT), lambda i: (i, 0)),
            pl.BlockSpec((1, D_OUT), lambda i: (0, 0)),
        ],
        out_specs=pl.BlockSpec((blk, D_OUT), lambda i: (i, 0)),
        out_shape=jax.ShapeDtypeStruct((N, D_OUT), jnp.float32),
    )(accp, g, dinvb, b)


def kernel(x, edge_index, W, b):
    h = _mm_call(x, W)
    accp, dinvb, g = _sc_kernel(edge_index, h)
    out = _fin_call(accp, g, dinvb, b.reshape(1, D_OUT))
    return out


# speedup vs baseline: 1.2054x; 1.0326x over previous
import functools

import jax
import jax.numpy as jnp
from jax import lax
from jax.experimental import pallas as pl
from jax.experimental.pallas import tpu as pltpu
from jax.experimental.pallas import tpu_sc as plsc

N = 10000
D_IN = 128
D_OUT = 16
E = 320000

NC = 2
NS = 16
NW = NC * NS
LANES = 16
NPAD = 10240
RPS = NPAD // NS
EPS = E // NS
EPW = E // NW
CH = 1000
KCH = EPW // CH
NBUF = 2

_mesh = plsc.VectorSubcoreMesh(
    core_axis_name="c", subcore_axis_name="s", num_cores=NC, num_subcores=NS
)
_CP2 = pltpu.CompilerParams(needs_layout_passes=False, use_tc_tiling_on_sc=False)


def _mm_body(x_ref, w_ref, h_ref):
    h_ref[...] = jnp.dot(x_ref[...], w_ref[...],
                         preferred_element_type=jnp.float32)


def _mm_call(x, w):
    blk = 1024
    grid = NPAD // blk
    return pl.pallas_call(
        _mm_body,
        grid=(grid,),
        in_specs=[
            pl.BlockSpec((blk, D_IN), lambda i: (i, 0)),
            pl.BlockSpec((D_IN, D_OUT), lambda i: (0, 0)),
        ],
        out_specs=pl.BlockSpec((blk, D_OUT), lambda i: (i, 0)),
        out_shape=jax.ShapeDtypeStruct((NPAD, D_OUT), jnp.float32),
    )(x, w)


def _newton_rsqrt(x):
    i = plsc.bitcast(x, jnp.int32)
    i = jnp.int32(0x5F3759DF) - lax.shift_right_arithmetic(i, 1)
    y = plsc.bitcast(i, jnp.float32)
    xh = x * 0.5
    for _ in range(3):
        y = y * (1.5 - xh * y * y)
    return y


@functools.partial(
    pl.kernel,
    out_type=(
        jax.ShapeDtypeStruct((NC, NPAD, D_OUT), jnp.float32),
        jax.ShapeDtypeStruct((NPAD, D_OUT), jnp.float32),
    ),
    mesh=_mesh,
    compiler_params=_CP2,
    scratch_types=[
        pltpu.VMEM((EPS,), jnp.int32),
        pltpu.VMEM((NPAD,), jnp.float32),
        pltpu.VMEM((NS, RPS), jnp.float32),
        pltpu.VMEM((RPS,), jnp.float32),
        pltpu.VMEM((RPS, D_OUT), jnp.float32),
        pltpu.VMEM((EPW,), jnp.int32),
        pltpu.VMEM((EPW,), jnp.int32),
        pltpu.VMEM((NBUF, CH, D_OUT), jnp.float32),
        pltpu.VMEM_SHARED((NS, NPAD), jnp.float32),
        pltpu.VMEM_SHARED((NPAD, D_OUT), jnp.float32),
        pltpu.SemaphoreType.DMA,
        pltpu.SemaphoreType.DMA,
    ],
)
def _sc_kernel(edge_hbm, h_hbm,
               acc_hbm, g_hbm,
               cidx_v, hist_v, tmp_v, dinv_v, buf_v,
               src_v, dstc_v, rows_v, pub_sh, acc_sh, sem0, sem1):
    cid = lax.axis_index("c")
    sid = lax.axis_index("s")
    wid = cid * NS + sid
    base = sid * RPS
    sems = (sem0, sem1)

    pltpu.sync_copy(edge_hbm.at[1, pl.ds(sid * EPS, EPS)], cidx_v)

    def _zero_hist(i, carry):
        hist_v[pl.ds(i * LANES, LANES)] = jnp.zeros((LANES,), jnp.float32)
        return carry
    lax.fori_loop(0, NPAD // LANES, _zero_hist, 0, unroll=8)

    ones = jnp.ones((LANES,), jnp.float32)

    def _count(i, carry):
        idx = cidx_v[pl.ds(i * LANES, LANES)]
        plsc.addupdate_scatter(hist_v, [idx], ones)
        return carry
    lax.fori_loop(0, EPS // LANES, _count, 0, unroll=8)

    pltpu.sync_copy(hist_v, pub_sh.at[sid])

    pltpu.sync_copy(edge_hbm.at[0, pl.ds(wid * EPW, EPW)], src_v)
    pltpu.sync_copy(edge_hbm.at[1, pl.ds(wid * EPW, EPW)], dstc_v)

    def _zero_buf(i, carry):
        buf_v[i] = jnp.zeros((D_OUT,), jnp.float32)
        return carry
    lax.fori_loop(0, RPS, _zero_buf, 0, unroll=8)
    pltpu.sync_copy(buf_v, acc_sh.at[pl.ds(base, RPS)])

    plsc.subcore_barrier()

    pltpu.sync_copy(pub_sh.at[:, pl.ds(base, RPS)], tmp_v)

    def _comb(j, carry):
        sl = pl.ds(j * LANES, LANES)
        v = tmp_v[0, sl]
        for t in range(1, NS):
            v = v + tmp_v[t, sl]
        dinv_v[sl] = _newton_rsqrt(v + 1.0)
        return carry
    lax.fori_loop(0, RPS // LANES, _comb, 0, unroll=4)

    pltpu.sync_copy(h_hbm.at[pl.ds(base, RPS)], buf_v)

    def _scale(r, carry):
        w = plsc.load_gather(dinv_v, [jnp.full((LANES,), r, jnp.int32)])
        buf_v[r] = buf_v[r] * w
        return carry
    lax.fori_loop(0, RPS, _scale, 0, unroll=8)

    pltpu.sync_copy(buf_v, g_hbm.at[pl.ds(base, RPS)])
    plsc.subcore_barrier()

    pltpu.async_copy(g_hbm.at[src_v.at[pl.ds(0, CH)]], rows_v.at[0], sem0)

    def _chunk(k, carry):
        p = lax.rem(k, NBUF)

        @pl.when(k + 1 < KCH)
        def _():
            pn = lax.rem(k + 1, NBUF)
            for q in range(NBUF):
                @pl.when(pn == q)
                def _():
                    pltpu.async_copy(
                        g_hbm.at[src_v.at[pl.ds((k + 1) * CH, CH)]],
                        rows_v.at[q], sems[q])

        for q in range(NBUF):
            @pl.when(p == q)
            def _():
                pltpu.make_async_copy(
                    g_hbm.at[src_v.at[pl.ds(k * CH, CH)]],
                    rows_v.at[q], sems[q]).wait()
                pltpu.sync_copy(rows_v.at[q],
                                acc_sh.at[dstc_v.at[pl.ds(k * CH, CH)]],
                                add=True)
        return carry
    lax.fori_loop(0, KCH, _chunk, 0)

    plsc.subcore_barrier()
    pltpu.sync_copy(acc_sh.at[pl.ds(base, RPS)], buf_v)

    @pl.when(cid == 0)
    def _():
        pltpu.sync_copy(g_hbm.at[pl.ds(base, RPS)], rows_v.at[0, pl.ds(0, RPS)])

        def _addg(r, carry):
            buf_v[r] = buf_v[r] + rows_v[0, r]
            return carry
        lax.fori_loop(0, RPS, _addg, 0, unroll=8)

    def _fscale(r, carry):
        w = plsc.load_gather(dinv_v, [jnp.full((LANES,), r, jnp.int32)])
        buf_v[r] = buf_v[r] * w
        return carry
    lax.fori_loop(0, RPS, _fscale, 0, unroll=8)
    pltpu.sync_copy(buf_v, acc_hbm.at[cid, pl.ds(base, RPS), :])


def _fin_body(accp_ref, b_ref, out_ref):
    out_ref[...] = accp_ref[0, :, :] + accp_ref[1, :, :] + b_ref[0, :]


def _fin_call(accp, b):
    blk = 1024
    grid = NPAD // blk
    return pl.pallas_call(
        _fin_body,
        grid=(grid,),
        in_specs=[
            pl.BlockSpec((NC, blk, D_OUT), lambda i: (0, i, 0)),
            pl.BlockSpec((1, D_OUT), lambda i: (0, 0)),
        ],
        out_specs=pl.BlockSpec((blk, D_OUT), lambda i: (i, 0)),
        out_shape=jax.ShapeDtypeStruct((N, D_OUT), jnp.float32),
    )(accp, b)


def kernel(x, edge_index, W, b):
    h = _mm_call(x, W)
    accp, g = _sc_kernel(edge_index, h)
    del g
    out = _fin_call(accp, b.reshape(1, D_OUT))
    return out
